# Initial kernel scaffold; baseline (speedup 1.0000x reference)
import jax
import jax.numpy as jnp
from jax.experimental import pallas as pl


def kernel(rays, depth, density):
    raise NotImplementedError("write your pallas kernel here")


# speedup vs baseline: 8.4352x; 8.4352x over previous
import jax
import jax.numpy as jnp
from jax.experimental import pallas as pl
from jax.experimental.pallas import tpu as pltpu

FN = 64
L = 64
RBLK = 1024


def _body(u_ref, depth_ref, dens_ref, rays_ref, t_ref, fsp_ref):
    R = depth_ref.shape[1]
    depth = depth_ref[...]
    dens = dens_ref[...]

    one_row = jnp.ones((1, R), jnp.float32)
    delta = jnp.concatenate(
        [depth[1:L] - depth[0:L - 1], jnp.full((1, R), 1e10, jnp.float32)], axis=0)
    neg_x = (-jnp.maximum(dens, 0.0)) * delta
    E = jnp.exp(neg_x)
    alpha = 1.0 - E
    terms = (1.0 - alpha) + 1e-10

    rows = [one_row]
    for i in range(1, L):
        rows.append(rows[-1] * terms[i - 1:i])
    trans = jnp.concatenate(rows, axis=0)

    wp = alpha * trans + 1e-5

    P = wp[1:9]
    for k in range(1, 7):
        P = P + wp[1 + 8 * k: 9 + 8 * k]
    P = P + jnp.concatenate([wp[57:64], jnp.zeros((1, R), jnp.float32)], axis=0)
    W = (((P[0:1] + P[4:5]) + (P[2:3] + P[6:7]))
         + ((P[1:2] + P[5:6]) + (P[3:4] + P[7:8])))

    inv_w = 1.0 / W
    pdf = wp[1:L] * inv_w

    crows = [jnp.zeros((1, R), jnp.float32), pdf[0:1]]
    for i in range(1, L - 1):
        crows.append(crows[-1] + pdf[i:i + 1])
    cdf_s = jnp.concatenate(crows, axis=0)

    dn = cdf_s[1:L] - cdf_s[0:L - 1]
    dn = jnp.where(dn < 1e-5, jnp.ones_like(dn), dn)
    S_lv = (depth[1:L] - depth[0:L - 1]) / dn

    u = u_ref[...][:, 0:1]

    B = jnp.broadcast_to(depth[0:1], (FN, R))
    C = jnp.zeros((FN, R), jnp.float32)
    S = jnp.broadcast_to(S_lv[0:1], (FN, R))
    for l in range(1, L - 1):
        m = cdf_s[l:l + 1, :] < u
        B = jnp.where(m, depth[l:l + 1], B)
        C = jnp.where(m, cdf_s[l:l + 1], C)
        S = jnp.where(m, S_lv[l:l + 1], S)
    m = cdf_s[L - 1:L, :] < u
    B = jnp.where(m, depth[0:1], B)
    C = jnp.where(m, jnp.zeros((1, R), jnp.float32), C)
    S = jnp.where(m, S_lv[0:1], S)

    t = B + S * (u - C)
    t_ref[...] = t
    fsp_ref[0] = t * rays_ref[3:4] + rays_ref[0:1]
    fsp_ref[1] = t * rays_ref[4:5] + rays_ref[1:2]
    fsp_ref[2] = t * rays_ref[5:6] + rays_ref[2:3]


def kernel(rays, depth, density):
    n = depth.shape[0]
    depth_t = depth[:, :, 0].T
    dens_t = density[:, :, 0].T
    rays_t = rays.T
    u = jnp.linspace(0.0, 1.0, FN, dtype=jnp.float32)
    u2 = jnp.broadcast_to(u.reshape(FN, 1), (FN, 128))

    grid = n // RBLK
    t_t, fsp_t = pl.pallas_call(
        _body,
        grid=(grid,),
        in_specs=[
            pl.BlockSpec((FN, 128), lambda i: (0, 0)),
            pl.BlockSpec((L, RBLK), lambda i: (0, i)),
            pl.BlockSpec((L, RBLK), lambda i: (0, i)),
            pl.BlockSpec((6, RBLK), lambda i: (0, i)),
        ],
        out_specs=[
            pl.BlockSpec((FN, RBLK), lambda i: (0, i)),
            pl.BlockSpec((3, FN, RBLK), lambda i: (0, 0, i)),
        ],
        out_shape=[
            jax.ShapeDtypeStruct((FN, n), jnp.float32),
            jax.ShapeDtypeStruct((3, FN, n), jnp.float32),
        ],
        compiler_params=pltpu.CompilerParams(
            dimension_semantics=("arbitrary",),
        ),
    )(u2, depth_t, dens_t, rays_t)
    return (t_t.T, jnp.transpose(fsp_t, (2, 1, 0)))


# speedup vs baseline: 11.0953x; 1.3153x over previous
import jax
import jax.numpy as jnp
from jax.experimental import pallas as pl
from jax.experimental.pallas import tpu as pltpu

FN = 64
L = 64
RBLK = 256


def _body(u_ref, depth_ref, dens_ref, rays_ref, t_ref, fsp_ref):
    R = depth_ref.shape[1]
    depth = depth_ref[...]
    dens = dens_ref[...]

    one_row = jnp.ones((1, R), jnp.float32)
    delta = jnp.concatenate(
        [depth[1:L] - depth[0:L - 1], jnp.full((1, R), 1e10, jnp.float32)], axis=0)
    neg_x = (-jnp.maximum(dens, 0.0)) * delta
    E = jnp.exp(neg_x)
    alpha = 1.0 - E
    terms = (1.0 - alpha) + 1e-10

    rows = [one_row]
    for i in range(1, L):
        rows.append(rows[-1] * terms[i - 1:i])
    trans = jnp.concatenate(rows, axis=0)

    wp = alpha * trans + 1e-5

    P = wp[1:9]
    for k in range(1, 7):
        P = P + wp[1 + 8 * k: 9 + 8 * k]
    P = P + jnp.concatenate([wp[57:64], jnp.zeros((1, R), jnp.float32)], axis=0)
    W = (((P[0:1] + P[4:5]) + (P[2:3] + P[6:7]))
         + ((P[1:2] + P[5:6]) + (P[3:4] + P[7:8])))

    inv_w = 1.0 / W
    pdf = wp[1:L] * inv_w

    crows = [jnp.zeros((1, R), jnp.float32), pdf[0:1]]
    for i in range(1, L - 1):
        crows.append(crows[-1] + pdf[i:i + 1])
    cdf_s = jnp.concatenate(crows, axis=0)

    dn = cdf_s[1:L] - cdf_s[0:L - 1]
    dn = jnp.where(dn < 1e-5, jnp.ones_like(dn), dn)
    S_lv = (depth[1:L] - depth[0:L - 1]) / dn
    A_lv = depth[0:L - 1] - S_lv * cdf_s[0:L - 1]

    u = u_ref[...][:, 0:1]

    A = jnp.broadcast_to(A_lv[0:1], (FN, R))
    S = jnp.broadcast_to(S_lv[0:1], (FN, R))
    for l in range(1, L - 1):
        m = cdf_s[l:l + 1, :] < u
        A = jnp.where(m, A_lv[l:l + 1], A)
        S = jnp.where(m, S_lv[l:l + 1], S)
    m = cdf_s[L - 1:L, :] < u
    A = jnp.where(m, A_lv[0:1], A)
    S = jnp.where(m, S_lv[0:1], S)

    t = A + S * u
    t_ref[...] = t
    fsp_ref[0] = t * rays_ref[3:4] + rays_ref[0:1]
    fsp_ref[1] = t * rays_ref[4:5] + rays_ref[1:2]
    fsp_ref[2] = t * rays_ref[5:6] + rays_ref[2:3]


def kernel(rays, depth, density):
    n = depth.shape[0]
    depth_t = depth[:, :, 0].T
    dens_t = density[:, :, 0].T
    rays_t = rays.T
    u = jnp.linspace(0.0, 1.0, FN, dtype=jnp.float32)
    u2 = jnp.broadcast_to(u.reshape(FN, 1), (FN, 128))

    grid = n // RBLK
    t_t, fsp_t = pl.pallas_call(
        _body,
        grid=(grid,),
        in_specs=[
            pl.BlockSpec((FN, 128), lambda i: (0, 0)),
            pl.BlockSpec((L, RBLK), lambda i: (0, i)),
            pl.BlockSpec((L, RBLK), lambda i: (0, i)),
            pl.BlockSpec((6, RBLK), lambda i: (0, i)),
        ],
        out_specs=[
            pl.BlockSpec((FN, RBLK), lambda i: (0, i)),
            pl.BlockSpec((3, FN, RBLK), lambda i: (0, 0, i)),
        ],
        out_shape=[
            jax.ShapeDtypeStruct((FN, n), jnp.float32),
            jax.ShapeDtypeStruct((3, FN, n), jnp.float32),
        ],
        compiler_params=pltpu.CompilerParams(
            dimension_semantics=("arbitrary",),
        ),
    )(u2, depth_t, dens_t, rays_t)
    return (t_t.T, jnp.transpose(fsp_t, (2, 1, 0)))
